# baseline (speedup 1.0000x reference)
import functools

import jax
import jax.numpy as jnp
from jax import lax
from jax.experimental import pallas as pl
from jax.experimental.pallas import tpu as pltpu
from jax.experimental.pallas import tpu_sc as plsc

NC, NS, L = 2, 16, 16
NW = NC * NS
B, S, D = 4, 4096, 1024
SCALE = 32.0
PW = S // NW
CP = 8
NCHUNK = PW // CP
NST = 3
GROUPS = D // L

_mesh = plsc.VectorSubcoreMesh(core_axis_name="c", subcore_axis_name="s")


@functools.partial(
    pl.kernel,
    out_type=jax.ShapeDtypeStruct((B, S, D), jnp.float32),
    mesh=_mesh,
    scratch_types=[
        pltpu.VMEM((B, NCHUNK, CP), jnp.int32),
        pltpu.VMEM((CP, D), jnp.float32),
        pltpu.VMEM((CP, D), jnp.float32),
        *[pltpu.VMEM((CP, D), jnp.float32) for _ in range(NST * B)],
        pltpu.SemaphoreType.DMA,
        pltpu.SemaphoreType.DMA,
        pltpu.SemaphoreType.DMA,
        *[pltpu.SemaphoreType.DMA for _ in range(NST * B)],
        *[pltpu.SemaphoreType.DMA for _ in range(NST * B)],
    ],
)
def _emb_kernel(inputs_hbm, tok_hbm, pos_hbm, out_hbm, idx_v, pos0, pos1,
                *rest):
    NBUF = NST * B
    tok = list(rest[:NBUF])
    si = rest[NBUF]
    sp = [rest[NBUF + 1], rest[NBUF + 2]]
    sg = list(rest[NBUF + 3:NBUF + 3 + NBUF])
    so = list(rest[NBUF + 3 + NBUF:NBUF + 3 + 2 * NBUF])
    posb = [pos0, pos1]

    wid = lax.axis_index("s") * NC + lax.axis_index("c")
    p0 = wid * PW

    hidx = []
    for b in range(B):
        h = pltpu.make_async_copy(
            inputs_hbm.at[b, pl.ds(pl.multiple_of(p0 // CP, NCHUNK), NCHUNK)],
            idx_v.at[b], si)
        h.start()
        hidx.append(h)

    def start_pos(c):
        h = pltpu.make_async_copy(
            pos_hbm.at[pl.ds(p0 + c * CP, CP)], posb[c % 2], sp[c % 2])
        h.start()
        return h

    def start_gather(c, b):
        k = (c % NST) * B + b
        h = pltpu.make_async_copy(tok_hbm.at[idx_v.at[b, c]], tok[k], sg[k])
        h.start()
        return h

    def start_out(c, b):
        k = (c % NST) * B + b
        h = pltpu.make_async_copy(
            tok[k], out_hbm.at[b, pl.ds(p0 + c * CP, CP)], so[k])
        h.start()
        return h

    for h in hidx:
        h.wait()

    hp = [start_pos(0), None]
    hg = [None] * NBUF
    ho = [None] * NBUF
    for b in range(B):
        hg[b] = start_gather(0, b)

    for c in range(NCHUNK):
        st = c % NST
        cn = c + 1
        if cn < NCHUNK:
            for b in range(B):
                k = (cn % NST) * B + b
                if ho[k] is not None:
                    ho[k].wait()
            hp[cn % 2] = start_pos(cn)
            for b in range(B):
                hg[(cn % NST) * B + b] = start_gather(cn, b)

        for b in range(B):
            hg[st * B + b].wait()
        hp[c % 2].wait()

        pbuf = posb[c % 2]
        t0, t1, t2, t3 = tok[st * B:st * B + B]

        @pl.loop(0, CP * GROUPS, unroll=4)
        def _fma(g):
            r = g // GROUPS
            off = (g % GROUPS) * L
            sl = pl.ds(off, L)
            p = pbuf[r, sl]
            t0[r, sl] = t0[r, sl] * SCALE + p
            t1[r, sl] = t1[r, sl] * SCALE + p
            t2[r, sl] = t2[r, sl] * SCALE + p
            t3[r, sl] = t3[r, sl] * SCALE + p

        for b in range(B):
            ho[st * B + b] = start_out(c, b)

    for k in range(NBUF):
        if ho[k] is not None:
            ho[k].wait()


def kernel(inputs, tok_table, pos_table):
    idx = inputs.astype(jnp.int32).reshape(B, S // CP, CP)
    return _emb_kernel(idx, tok_table, pos_table)


# speedup vs baseline: 1.0341x; 1.0341x over previous
import functools

import jax
import jax.numpy as jnp
from jax import lax
from jax.experimental import pallas as pl
from jax.experimental.pallas import tpu as pltpu
from jax.experimental.pallas import tpu_sc as plsc

NC, NS, L = 2, 16, 16
NW = NC * NS
B, S, D = 4, 4096, 1024
SCALE = 32.0
PW = S // NW
CP = 16
NCHUNK = PW // CP
NJ = NCHUNK * B
NB = 5
AH = 3
GROUPS = D // L

_mesh = plsc.VectorSubcoreMesh(core_axis_name="c", subcore_axis_name="s")


@functools.partial(
    pl.kernel,
    out_type=jax.ShapeDtypeStruct((B, S, D), jnp.float32),
    mesh=_mesh,
    scratch_types=[
        pltpu.VMEM((B, NCHUNK, CP), jnp.int32),
        pltpu.VMEM((CP, D), jnp.float32),
        pltpu.VMEM((CP, D), jnp.float32),
        *[pltpu.VMEM((CP, D), jnp.float32) for _ in range(NB)],
        pltpu.SemaphoreType.DMA,
        pltpu.SemaphoreType.DMA,
        pltpu.SemaphoreType.DMA,
        *[pltpu.SemaphoreType.DMA for _ in range(NB)],
        *[pltpu.SemaphoreType.DMA for _ in range(NB)],
    ],
)
def _emb_kernel(inputs_hbm, tok_hbm, pos_hbm, out_hbm, idx_v, pos0, pos1,
                *rest):
    tok = list(rest[:NB])
    si = rest[NB]
    sp = [rest[NB + 1], rest[NB + 2]]
    sg = list(rest[NB + 3:NB + 3 + NB])
    so = list(rest[NB + 3 + NB:NB + 3 + 2 * NB])
    posb = [pos0, pos1]

    wid = lax.axis_index("s") * NC + lax.axis_index("c")
    p0 = wid * PW

    hidx = []
    for b in range(B):
        h = pltpu.make_async_copy(
            inputs_hbm.at[b, pl.ds(pl.multiple_of(p0 // CP, NCHUNK), NCHUNK)],
            idx_v.at[b], si)
        h.start()
        hidx.append(h)
    idx_ready = [False] * B

    def start_pos(c):
        h = pltpu.make_async_copy(
            pos_hbm.at[pl.ds(p0 + c * CP, CP)], posb[c % 2], sp[c % 2])
        h.start()
        return h

    def start_gather(j):
        c, b = j // B, j % B
        nb = j % NB
        if not idx_ready[b]:
            hidx[b].wait()
            idx_ready[b] = True
        h = pltpu.make_async_copy(
            tok_hbm.at[idx_v.at[b, c]], tok[nb], sg[nb])
        h.start()
        return h

    def start_out(j):
        c, b = j // B, j % B
        nb = j % NB
        h = pltpu.make_async_copy(
            tok[nb], out_hbm.at[b, pl.ds(p0 + c * CP, CP)], so[nb])
        h.start()
        return h

    hp = [start_pos(0), start_pos(1)]
    hg = [None] * NB
    ho = [None] * NB
    for k in range(AH):
        hg[k % NB] = start_gather(k)

    for j in range(NJ):
        c, b = j // B, j % B
        nb = j % NB
        jn = j + AH
        if jn < NJ:
            cn, bn = jn // B, jn % B
            tb = jn % NB
            if ho[tb] is not None:
                ho[tb].wait()
            if bn == 0 and cn >= 2:
                hp[cn % 2] = start_pos(cn)
            hg[tb] = start_gather(jn)

        hg[nb].wait()
        if b == 0:
            hp[c % 2].wait()

        tbuf = tok[nb]
        pbuf = posb[c % 2]

        @pl.loop(0, CP * GROUPS, unroll=8)
        def _fma(g):
            r = g // GROUPS
            off = (g % GROUPS) * L
            t = tbuf[r, pl.ds(off, L)]
            p = pbuf[r, pl.ds(off, L)]
            tbuf[r, pl.ds(off, L)] = t * SCALE + p

        ho[nb] = start_out(j)

    for nb in range(NB):
        if ho[nb] is not None:
            ho[nb].wait()


def kernel(inputs, tok_table, pos_table):
    idx = inputs.astype(jnp.int32).reshape(B, S // CP, CP)
    return _emb_kernel(idx, tok_table, pos_table)
